# Initial kernel scaffold; baseline (speedup 1.0000x reference)
import jax
import jax.numpy as jnp
from jax.experimental import pallas as pl


def kernel(part_pcs, n_pcs, W, b):
    raise NotImplementedError("write your pallas kernel here")


# speedup vs baseline: 2.0226x; 2.0226x over previous
import functools

import jax
import jax.numpy as jnp
import numpy as np
from jax import lax
from jax.experimental import pallas as pl
from jax.experimental.pallas import tpu as pltpu

B, N_SUM, P = 4, 2048, 64
NUM_BASES = 16
DIST_LO, DIST_HI = 0.0, 10.0
ANG_LO, ANG_HI = -1.0, 1.0

ROW_TILE = 256
T = N_SUM // ROW_TILE


def _pair_kernel(pcs_ref, n_ref, w_ref, b_ref, out_ref, oh_s, c_s):
    t = pl.program_id(1)

    @pl.when(t == 0)
    def _compute_pair_stage():
        counts = n_ref[0].astype(jnp.float32)
        iota_q = lax.broadcasted_iota(jnp.int32, (P, P), 0).astype(jnp.float32)
        iota_p = lax.broadcasted_iota(jnp.int32, (P, P), 1).astype(jnp.float32)
        lt = (iota_q <= iota_p).astype(jnp.float32)
        cum = jnp.dot(counts, lt, preferred_element_type=jnp.float32)
        total = cum[:, P - 1 :]

        iota_n = lax.broadcasted_iota(jnp.int32, (N_SUM, P), 0).astype(jnp.float32)
        cum_b = jnp.broadcast_to(cum, (N_SUM, P))
        pp = jnp.sum((iota_n >= cum_b).astype(jnp.float32), axis=1, keepdims=True)
        valid = (iota_n[:, :1] < total).astype(jnp.float32)
        pidx = jnp.where(valid > 0.0, jnp.minimum(pp, float(P - 1)), 0.0)
        part_iota = lax.broadcasted_iota(jnp.int32, (N_SUM, P), 1).astype(jnp.float32)
        oh = (pidx == part_iota).astype(jnp.float32)
        oh_s[...] = oh

        ohv = oh * valid
        pcs = pcs_ref[0]
        sums = jax.lax.dot_general(
            pcs, ohv, (((1,), (0,)), ((), ())),
            preferred_element_type=jnp.float32,
        )
        cent = sums / jnp.maximum(counts, 1.0)

        offdiag = (iota_q != iota_p).astype(jnp.float32)
        diff = (cent[:, :, None] - cent[:, None, :]) * offdiag[None, :, :]
        d2 = jnp.sum(diff * diff, axis=0)
        distances = jnp.sqrt(jnp.maximum(d2, 1e-24))
        vnorm = jnp.maximum(jnp.sqrt(jnp.maximum(d2, 1e-16)), 1e-8)
        vhat = diff / vnorm[None, :, :]

        wd = (DIST_HI - DIST_LO) / (NUM_BASES - 1)
        cd = np.float32(-0.5 / (wd * wd))
        offs_d = (
            lax.broadcasted_iota(jnp.int32, (P, P, NUM_BASES), 2).astype(jnp.float32)
            * np.float32(wd) + np.float32(DIST_LO)
        )
        ed = jnp.exp(cd * (distances[:, :, None] - offs_d) ** 2)
        w_dist = w_ref[0, :NUM_BASES]
        dist_c = jax.lax.dot_general(
            ed.reshape(P * P, NUM_BASES), w_dist.reshape(NUM_BASES, 1),
            (((1,), (0,)), ((), ())), preferred_element_type=jnp.float32,
        ).reshape(P, P)

        cos3 = vhat[0][:, :, None] * vhat[0][:, None, :]
        cos3 += vhat[1][:, :, None] * vhat[1][:, None, :]
        cos3 += vhat[2][:, :, None] * vhat[2][:, None, :]
        cos3 = jnp.clip(cos3, -1.0, 1.0)

        wa_ = (ANG_HI - ANG_LO) / (NUM_BASES - 1)
        ca = np.float32(-0.5 / (wa_ * wa_))
        w_ang = w_ref[0, NUM_BASES:].reshape(NUM_BASES, 1)
        kmask = (counts > 0.0).astype(jnp.float32)
        CH = 8
        s_rows = []
        for ci in range(P // CH):
            sub = cos3[ci * CH : (ci + 1) * CH]
            offs_a = (
                lax.broadcasted_iota(jnp.int32, (CH, P, P, NUM_BASES), 3).astype(jnp.float32)
                * np.float32(wa_) + np.float32(ANG_LO)
            )
            ea = jnp.exp(ca * (sub[..., None] - offs_a) ** 2)
            f = jax.lax.dot_general(
                ea.reshape(CH * P * P, NUM_BASES), w_ang,
                (((1,), (0,)), ((), ())), preferred_element_type=jnp.float32,
            ).reshape(CH, P, P)
            s_rows.append(jnp.sum(f * kmask[None, :, :], axis=2))
        s = jnp.concatenate(s_rows, axis=0)

        ident = (iota_q == iota_p).astype(jnp.float32)
        s_t = jax.lax.dot_general(
            s, ident, (((0,), (0,)), ((), ())),
            preferred_element_type=jnp.float32,
        )
        pair_bias = dist_c + s_t + b_ref[0, 0]

        c_s[...] = jax.lax.dot_general(
            pair_bias, oh, (((1,), (1,)), ((), ())),
            preferred_element_type=jnp.float32,
        )

    rows = oh_s[pl.ds(t * ROW_TILE, ROW_TILE), :]
    out_ref[0, 0] = jnp.dot(rows, c_s[...], preferred_element_type=jnp.float32)


@jax.jit
def kernel(part_pcs, n_pcs, W, b):
    pcs_t = part_pcs.transpose(0, 2, 1)
    n3 = n_pcs.reshape(B, 1, P)
    b2 = b.reshape(1, 1)
    out = pl.pallas_call(
        _pair_kernel,
        grid=(B, T),
        in_specs=[
            pl.BlockSpec((1, 3, N_SUM), lambda bb, t: (bb, 0, 0)),
            pl.BlockSpec((1, 1, P), lambda bb, t: (bb, 0, 0)),
            pl.BlockSpec((1, NUM_BASES * 2), lambda bb, t: (0, 0)),
            pl.BlockSpec(memory_space=pltpu.SMEM),
        ],
        out_specs=pl.BlockSpec(
            (1, 1, ROW_TILE, N_SUM), lambda bb, t: (bb, 0, t, 0)
        ),
        out_shape=jax.ShapeDtypeStruct((B, 1, N_SUM, N_SUM), jnp.float32),
        scratch_shapes=[
            pltpu.VMEM((N_SUM, P), jnp.float32),
            pltpu.VMEM((P, N_SUM), jnp.float32),
        ],
        compiler_params=pltpu.CompilerParams(
            dimension_semantics=("arbitrary", "arbitrary"),
        ),
    )(pcs_t, n3, W, b2)
    return out


# speedup vs baseline: 13.3506x; 6.6006x over previous
import functools

import jax
import jax.numpy as jnp
import numpy as np
from jax import lax
from jax.experimental import pallas as pl
from jax.experimental.pallas import tpu as pltpu

B, N_SUM, P = 4, 2048, 64
NUM_BASES = 16
DIST_LO, DIST_HI = 0.0, 10.0
ANG_LO, ANG_HI = -1.0, 1.0

ROW_TILE = 256
T = N_SUM // ROW_TILE


def _pair_kernel(pcs_ref, n_ref, w_ref, b_ref, out_ref, oh_s, c_s):
    t = pl.program_id(1)

    @pl.when(t == 0)
    def _compute_pair_stage():
        counts = n_ref[0].astype(jnp.float32)
        iota_q = lax.broadcasted_iota(jnp.int32, (P, P), 0).astype(jnp.float32)
        iota_p = lax.broadcasted_iota(jnp.int32, (P, P), 1).astype(jnp.float32)
        lt = (iota_q <= iota_p).astype(jnp.float32)
        cum = jnp.dot(counts, lt, preferred_element_type=jnp.float32)
        total = cum[:, P - 1 :]

        iota_n = lax.broadcasted_iota(jnp.int32, (N_SUM, P), 0).astype(jnp.float32)
        cum_b = jnp.broadcast_to(cum, (N_SUM, P))
        pp = jnp.sum((iota_n >= cum_b).astype(jnp.float32), axis=1, keepdims=True)
        valid = (iota_n[:, :1] < total).astype(jnp.float32)
        pidx = jnp.where(valid > 0.0, jnp.minimum(pp, float(P - 1)), 0.0)
        part_iota = lax.broadcasted_iota(jnp.int32, (N_SUM, P), 1).astype(jnp.float32)
        oh = (pidx == part_iota).astype(jnp.float32)
        oh_s[...] = oh

        ohv = oh * valid
        pcs = pcs_ref[0]
        sums = jax.lax.dot_general(
            pcs, ohv, (((1,), (0,)), ((), ())),
            preferred_element_type=jnp.float32,
        )
        cent = sums / jnp.maximum(counts, 1.0)

        offdiag = (iota_q != iota_p).astype(jnp.float32)
        diff = (cent[:, :, None] - cent[:, None, :]) * offdiag[None, :, :]
        d2 = jnp.sum(diff * diff, axis=0)
        distances = jnp.sqrt(jnp.maximum(d2, 1e-24))
        vnorm = jnp.maximum(jnp.sqrt(jnp.maximum(d2, 1e-16)), 1e-8)
        vhat = diff / vnorm[None, :, :]

        wd = (DIST_HI - DIST_LO) / (NUM_BASES - 1)
        cd = np.float32(-0.5 / (wd * wd))
        dist_c = jnp.zeros((P, P), jnp.float32)
        for base in range(NUM_BASES):
            off = np.float32(DIST_LO + wd * base)
            dd = distances - off
            dist_c = dist_c + w_ref[0, base] * jnp.exp(cd * dd * dd)

        cos3 = vhat[0][:, :, None] * vhat[0][:, None, :]
        cos3 += vhat[1][:, :, None] * vhat[1][:, None, :]
        cos3 += vhat[2][:, :, None] * vhat[2][:, None, :]
        cos3 = jnp.clip(cos3, -1.0, 1.0)

        wa_ = (ANG_HI - ANG_LO) / (NUM_BASES - 1)
        ca = np.float32(-0.5 / (wa_ * wa_))
        facc = jnp.zeros((P, P, P), jnp.float32)
        for base in range(NUM_BASES):
            off = np.float32(ANG_LO + wa_ * base)
            da = cos3 - off
            facc = facc + w_ref[0, NUM_BASES + base] * jnp.exp(ca * da * da)

        kmask = (counts > 0.0).astype(jnp.float32)
        s = jnp.sum(facc * kmask[None, :, :], axis=2)

        ident = (iota_q == iota_p).astype(jnp.float32)
        s_t = jax.lax.dot_general(
            s, ident, (((0,), (0,)), ((), ())),
            preferred_element_type=jnp.float32,
        )
        pair_bias = dist_c + s_t + b_ref[0, 0]

        c_s[...] = jax.lax.dot_general(
            pair_bias, oh, (((1,), (1,)), ((), ())),
            preferred_element_type=jnp.float32,
        )

    rows = oh_s[pl.ds(t * ROW_TILE, ROW_TILE), :]
    out_ref[0, 0] = jnp.dot(rows, c_s[...], preferred_element_type=jnp.float32)


@jax.jit
def kernel(part_pcs, n_pcs, W, b):
    pcs_t = part_pcs.transpose(0, 2, 1)
    n3 = n_pcs.reshape(B, 1, P)
    b2 = b.reshape(1, 1)
    out = pl.pallas_call(
        _pair_kernel,
        grid=(B, T),
        in_specs=[
            pl.BlockSpec((1, 3, N_SUM), lambda bb, t: (bb, 0, 0)),
            pl.BlockSpec((1, 1, P), lambda bb, t: (bb, 0, 0)),
            pl.BlockSpec(memory_space=pltpu.SMEM),
            pl.BlockSpec(memory_space=pltpu.SMEM),
        ],
        out_specs=pl.BlockSpec(
            (1, 1, ROW_TILE, N_SUM), lambda bb, t: (bb, 0, t, 0)
        ),
        out_shape=jax.ShapeDtypeStruct((B, 1, N_SUM, N_SUM), jnp.float32),
        scratch_shapes=[
            pltpu.VMEM((N_SUM, P), jnp.float32),
            pltpu.VMEM((P, N_SUM), jnp.float32),
        ],
        compiler_params=pltpu.CompilerParams(
            dimension_semantics=("arbitrary", "arbitrary"),
        ),
    )(pcs_t, n3, W, b2)
    return out
